# baseline (speedup 1.0000x reference)
import functools

import jax
import jax.numpy as jnp
from jax import lax
from jax.experimental import pallas as pl
from jax.experimental.pallas import tpu as pltpu
from jax.experimental.pallas import tpu_sc as plsc

BATCH = 16384
EMBED = 32

_info = plsc.get_sparse_core_info()
NC = _info.num_cores
NS = _info.num_subcores
LANES = _info.num_lanes
NW = NC * NS
BPW = BATCH // NW
CHUNK = 128
NCHUNK = BPW // CHUNK


def _gmf_body(users_hbm, items_hbm, utab_hbm, itab_hbm, out_hbm,
              idx_u, idx_i, urows, irows, orows, sem):
    wid = lax.axis_index("s") * NC + lax.axis_index("c")
    row0 = wid * NCHUNK

    pltpu.sync_copy(users_hbm.at[pl.ds(row0, NCHUNK)], idx_u)
    pltpu.sync_copy(items_hbm.at[pl.ds(row0, NCHUNK)], idx_i)

    copies = []
    for j in range(NCHUNK):
        copies.append(pltpu.async_copy(
            utab_hbm.at[idx_u.at[j]], urows.at[pl.ds(j * CHUNK, CHUNK)], sem))
        copies.append(pltpu.async_copy(
            itab_hbm.at[idx_i.at[j]], irows.at[pl.ds(j * CHUNK, CHUNK)], sem))
    for c in copies:
        c.wait()

    def mul_row(b, carry):
        orows[b, pl.ds(0, LANES)] = urows[b, pl.ds(0, LANES)] * irows[b, pl.ds(0, LANES)]
        orows[b, pl.ds(LANES, LANES)] = urows[b, pl.ds(LANES, LANES)] * irows[b, pl.ds(LANES, LANES)]
        return carry
    lax.fori_loop(0, BPW, mul_row, 0)

    pltpu.sync_copy(orows, out_hbm.at[pl.ds(wid * BPW, BPW)])


def kernel(users, items, user_table, item_table):
    users2 = users.astype(jnp.int32).reshape(NW * NCHUNK, CHUNK)
    items2 = items.astype(jnp.int32).reshape(NW * NCHUNK, CHUNK)
    run = pl.kernel(
        _gmf_body,
        out_type=jax.ShapeDtypeStruct((BATCH, EMBED), jnp.float32),
        mesh=plsc.VectorSubcoreMesh(core_axis_name="c", subcore_axis_name="s"),
        scratch_types=[
            pltpu.VMEM((NCHUNK, CHUNK), jnp.int32),
            pltpu.VMEM((NCHUNK, CHUNK), jnp.int32),
            pltpu.VMEM((BPW, EMBED), jnp.float32),
            pltpu.VMEM((BPW, EMBED), jnp.float32),
            pltpu.VMEM((BPW, EMBED), jnp.float32),
            pltpu.SemaphoreType.DMA,
        ],
        compiler_params=pltpu.CompilerParams(use_tc_tiling_on_sc=False),
    )
    return run(users2, items2, user_table, item_table)
